# baseline (speedup 1.0000x reference)
import functools

import jax
import jax.numpy as jnp
from jax import lax
from jax.experimental import pallas as pl
from jax.experimental.pallas import tpu as pltpu
from jax.experimental.pallas import tpu_sc as plsc

_V = 1000000
_D = 32
_B = 16384

_NW = 32
_SPAN = 31232
_WIN = 512
_NWIN = _SPAN // _WIN
_ROWCAP = 128
_DUMMY0 = _B
_XCHUNK = 2048
_WTRASH = _B + 16
_NBIN = 64

_mesh = plsc.VectorSubcoreMesh(core_axis_name="c", subcore_axis_name="s")


@functools.partial(
    pl.kernel,
    mesh=_mesh,
    out_type=jax.ShapeDtypeStruct((_B + 32, 128), jnp.float32),
    scratch_types=[
        pltpu.VMEM((_XCHUNK,), jnp.int32),
        pltpu.VMEM((_B + 32,), jnp.int32),
        pltpu.VMEM((_B + 32,), jnp.int32),
        pltpu.VMEM((_B + 32,), jnp.int32),
        pltpu.VMEM((_B + 32,), jnp.int32),
        pltpu.VMEM((_NBIN,), jnp.int32),
        pltpu.VMEM((_NBIN,), jnp.int32),
        pltpu.VMEM((_NBIN,), jnp.int32),
        pltpu.VMEM((_NBIN,), jnp.int32),
        pltpu.VMEM((32, _WIN), jnp.float32),
        pltpu.VMEM((32, _WIN), jnp.float32),
        pltpu.VMEM((_ROWCAP + 1, 128), jnp.float32),
        pltpu.VMEM((_ROWCAP + 16,), jnp.int32),
        pltpu.SemaphoreType.DMA,
        pltpu.SemaphoreType.DMA,
        pltpu.SemaphoreType.DMA,
    ],
    compiler_params=pltpu.CompilerParams(needs_layout_passes=False),
)
def _sc_scan_gather(
    x_hbm, emdT_hbm, tailT_hbm, out_hbm,
    xc_v, sel_r, sel_pos, wl_r, wl_pos, hist, offs, fill, probe,
    win0, win1, rowbuf, posbuf,
    sem0, sem1, sem_out,
):
    wid = lax.axis_index("s") * 2 + lax.axis_index("c")
    lo = wid * _SPAN
    hi = jnp.where(wid == _NW - 1, _V, lo + _SPAN)
    lane = lax.iota(jnp.int32, 16)
    dummy = _DUMMY0 + wid
    ones = jnp.ones((16,), jnp.int32)

    def win_start(w):
        return lo + w * _WIN

    def fire(w, buf, sem):
        off = pl.multiple_of(win_start(w), _WIN)
        pltpu.make_async_copy(
            emdT_hbm.at[:, pl.ds(off, _WIN)], buf, sem
        ).start()

    fire(0, win0, sem0)
    fire(1, win1, sem1)

    def chunk_body(cb, off_vec):
        pltpu.sync_copy(x_hbm.at[pl.ds(cb * _XCHUNK, _XCHUNK)], xc_v)

        def select_body(k, off_vec):
            iv = xc_v[pl.ds(k * 16, 16)]
            m = jnp.logical_and(iv >= lo, iv < hi)
            mi = jnp.where(m, 1, 0).astype(jnp.int32)
            prefix = plsc.cumsum(mi)
            slots = jnp.where(m, off_vec + prefix - 1, _WTRASH)
            plsc.store_scatter(sel_r, [slots], iv)
            plsc.store_scatter(
                sel_pos, [slots], lane + (cb * _XCHUNK + k * 16)
            )
            return off_vec + plsc.all_reduce_population_count(m)

        return lax.fori_loop(0, _XCHUNK // 16, select_body, off_vec)

    off_vec = lax.fori_loop(
        0, _B // _XCHUNK, chunk_body, jnp.zeros((16,), jnp.int32)
    )
    n_sel = jax.lax.shift_right_logical(jnp.sum(off_vec), 4)
    sel_r[pl.ds(n_sel, 16)] = jnp.full((16,), -1, jnp.int32)
    n_vreg = (n_sel + 15) // 16

    def window_of(rv):
        w = jax.lax.shift_right_logical(jnp.maximum(rv - lo, 0), 9)
        w = jnp.minimum(w, _NWIN)
        w = jnp.where(rv >= _V - 128, _NWIN + 1, w)
        return jnp.where(rv >= lo, w, _NBIN - 1)

    for k in range(_NBIN // 16):
        hist[pl.ds(k * 16, 16)] = jnp.zeros((16,), jnp.int32)

    def hist_body(v, _):
        rv = sel_r[pl.ds(v * 16, 16)]
        plsc.addupdate_scatter(hist, [window_of(rv)], ones)
        return 0

    lax.fori_loop(0, n_vreg, hist_body, 0)

    carry = jnp.zeros((16,), jnp.int32)
    for k in range(_NBIN // 16):
        h = hist[pl.ds(k * 16, 16)]
        incl = plsc.cumsum(h) + carry
        offs[pl.ds(k * 16, 16)] = incl - h
        fill[pl.ds(k * 16, 16)] = incl - h
        carry = plsc.load_gather(offs, [jnp.full((16,), k * 16 + 15, jnp.int32)]) \
            + plsc.load_gather(hist, [jnp.full((16,), k * 16 + 15, jnp.int32)])

    def perm_body(v, _):
        rv = sel_r[pl.ds(v * 16, 16)]
        posv = sel_pos[pl.ds(v * 16, 16)]
        w = window_of(rv)
        placed = jnp.zeros((16,), jnp.int32)
        for _rnd in range(16):
            pw = jnp.where(placed == 0, w, _NBIN - 1)
            plsc.store_scatter(probe, [pw], lane)
            back = plsc.load_gather(probe, [pw])
            winner = jnp.logical_and(back == lane, placed == 0)
            slotv = plsc.load_gather(fill, [pw])
            dst = jnp.where(winner, slotv, _WTRASH)
            plsc.store_scatter(wl_r, [dst], rv)
            plsc.store_scatter(wl_pos, [dst], posv)
            plsc.addupdate_scatter(fill, [pw], jnp.where(winner, 1, 0))
            placed = placed + jnp.where(winner, 1, 0)
        return 0

    lax.fori_loop(0, n_vreg, perm_body, 0)

    for k in range(_ROWCAP // 16):
        posbuf[pl.ds(k * 16, 16)] = jnp.full((16,), dummy, jnp.int32)

    def flush(off2):
        pltpu.make_async_copy(
            rowbuf.at[pl.ds(0, _ROWCAP)],
            out_hbm.at[posbuf.at[pl.ds(0, _ROWCAP)]],
            sem_out,
        ).start()
        pltpu.make_async_copy(
            rowbuf.at[pl.ds(0, _ROWCAP)],
            out_hbm.at[posbuf.at[pl.ds(0, _ROWCAP)]],
            sem_out,
        ).wait()
        for k in range(_ROWCAP // 16):
            posbuf[pl.ds(k * 16, 16)] = jnp.full((16,), dummy, jnp.int32)
        return 0

    def scalar_of(ref, i):
        s = plsc.load_gather(ref, [jnp.full((16,), i, jnp.int32)])
        return jax.lax.shift_right_logical(jnp.sum(s), 4)

    def extract_window(widx, w0, buf, off2):
        start = scalar_of(offs, widx)
        cnt = scalar_of(hist, widx)
        t0 = jax.lax.shift_right_logical(start, 4)
        t1 = jax.lax.shift_right_logical(start + cnt + 15, 4)

        def extract(t, off2):
            g = lane + t * 16
            m3 = jnp.logical_and(g >= start, g < start + cnt)
            rv = wl_r[pl.ds(t * 16, 16)]
            rloc = jnp.clip(rv - w0, 0, _WIN - 1)
            posv = wl_pos[pl.ds(t * 16, 16)]
            mi = jnp.where(m3, 1, 0).astype(jnp.int32)
            slots = jnp.where(m3, off2 + plsc.cumsum(mi) - 1, _ROWCAP)
            for c in range(_D):
                csplat = jnp.full((16,), c, jnp.int32)
                vals = plsc.load_gather(buf, [csplat, rloc])
                plsc.store_scatter(rowbuf, [slots, csplat], vals)
            plsc.store_scatter(posbuf, [slots], posv)
            off2 = off2 + jnp.sum(mi)

            @pl.when(off2 >= _ROWCAP - 16)
            def _():
                flush(off2)

            return jnp.where(off2 >= _ROWCAP - 16, 0, off2)

        return lax.fori_loop(t0, t1, extract, off2)

    def pair_body(j, off2):
        w_a = 2 * j
        w_b = 2 * j + 1
        pltpu.make_async_copy(
            emdT_hbm.at[:, pl.ds(pl.multiple_of(win_start(w_a), _WIN), _WIN)],
            win0, sem0,
        ).wait()
        off2 = extract_window(w_a, win_start(w_a), win0, off2)

        @pl.when(w_a + 2 < _NWIN)
        def _():
            fire(w_a + 2, win0, sem0)

        pltpu.make_async_copy(
            emdT_hbm.at[:, pl.ds(pl.multiple_of(win_start(w_b), _WIN), _WIN)],
            win1, sem1,
        ).wait()
        off2 = extract_window(w_b, win_start(w_b), win1, off2)

        @pl.when(w_b + 2 < _NWIN)
        def _():
            fire(w_b + 2, win1, sem1)

        return off2

    off2 = lax.fori_loop(0, _NWIN // 2, pair_body, 0)
    pltpu.make_async_copy(
        emdT_hbm.at[:, pl.ds(pl.multiple_of(win_start(_NWIN - 1), _WIN), _WIN)],
        win0, sem0,
    ).wait()
    off2 = extract_window(_NWIN - 1, win_start(_NWIN - 1), win0, off2)

    @pl.when(wid == _NW - 1)
    def _():
        pltpu.make_async_copy(
            emdT_hbm.at[:, pl.ds(999424, _WIN)], win0, sem0
        ).start()
        pltpu.make_async_copy(
            emdT_hbm.at[:, pl.ds(999424, _WIN)], win0, sem0
        ).wait()
        o = extract_window(_NWIN, 999424, win0, off2)
        pltpu.sync_copy(tailT_hbm, win0.at[:, pl.ds(0, 128)])
        o = extract_window(_NWIN + 1, _V - 128, win0, o)
        flush(o)

    @pl.when(wid != _NW - 1)
    def _():
        flush(off2)


def kernel(x, emd):
    emd_t = emd.T
    tail_t = lax.slice(emd_t, (0, _V - 128), (_D, _V))
    out_pad = _sc_scan_gather(x, emd_t, tail_t)
    return out_pad[:_B, :_D]


# speedup vs baseline: 1.0464x; 1.0464x over previous
import functools

import jax
import jax.numpy as jnp
from jax import lax
from jax.experimental import pallas as pl
from jax.experimental.pallas import tpu as pltpu
from jax.experimental.pallas import tpu_sc as plsc

_V = 1000000
_D = 32
_B = 16384

_NW = 32
_SPAN = 31232
_WIN = 512
_NWIN = _SPAN // _WIN
_ROWCAP = 128
_DUMMY0 = _B
_XCHUNK = 2048

_mesh = plsc.VectorSubcoreMesh(core_axis_name="c", subcore_axis_name="s")


@functools.partial(
    pl.kernel,
    mesh=_mesh,
    out_type=jax.ShapeDtypeStruct((_B + 32, 128), jnp.float32),
    scratch_types=[
        pltpu.VMEM((_XCHUNK,), jnp.int32),
        pltpu.VMEM((_B + 32,), jnp.int32),
        pltpu.VMEM((_B + 32,), jnp.int32),
        pltpu.VMEM((_B + 32,), jnp.int32),
        pltpu.VMEM((_B + 32,), jnp.int32),
        pltpu.VMEM((32, _WIN), jnp.float32),
        pltpu.VMEM((32, _WIN), jnp.float32),
        pltpu.VMEM((_ROWCAP + 1, 128), jnp.float32),
        pltpu.VMEM((_ROWCAP + 16,), jnp.int32),
        pltpu.SemaphoreType.DMA,
        pltpu.SemaphoreType.DMA,
        pltpu.SemaphoreType.DMA,
    ],
    compiler_params=pltpu.CompilerParams(needs_layout_passes=False),
)
def _sc_scan_gather(
    x_hbm, emdT_hbm, tailT_hbm, out_hbm,
    xc_v, sel_r, sel_pos, wl_r, wl_pos, win0, win1, rowbuf, posbuf,
    sem0, sem1, sem_out,
):
    wid = lax.axis_index("s") * 2 + lax.axis_index("c")
    lo = wid * _SPAN
    hi = jnp.where(wid == _NW - 1, _V, lo + _SPAN)
    lane = lax.iota(jnp.int32, 16)
    dummy = _DUMMY0 + wid

    def win_start(w):
        return lo + w * _WIN

    def fire(w, buf, sem):
        off = pl.multiple_of(win_start(w), _WIN)
        pltpu.make_async_copy(
            emdT_hbm.at[:, pl.ds(off, _WIN)], buf, sem
        ).start()

    fire(0, win0, sem0)
    fire(1, win1, sem1)

    def chunk_body(cb, off_vec):
        pltpu.sync_copy(x_hbm.at[pl.ds(cb * _XCHUNK, _XCHUNK)], xc_v)

        def select_body(k, off_vec):
            iv = xc_v[pl.ds(k * 16, 16)]
            m = jnp.logical_and(iv >= lo, iv < hi)
            mi = jnp.where(m, 1, 0).astype(jnp.int32)
            prefix = plsc.cumsum(mi)
            slots = jnp.where(m, off_vec + prefix - 1, _B + 16)
            plsc.store_scatter(sel_r, [slots], iv)
            plsc.store_scatter(
                sel_pos, [slots], lane + (cb * _XCHUNK + k * 16)
            )
            return off_vec + plsc.all_reduce_population_count(m)

        return lax.fori_loop(0, _XCHUNK // 16, select_body, off_vec)

    off_vec = lax.fori_loop(
        0, _B // _XCHUNK, chunk_body, jnp.zeros((16,), jnp.int32)
    )
    n_sel = jax.lax.shift_right_logical(jnp.sum(off_vec), 4)
    sel_r[pl.ds(n_sel, 16)] = jnp.full((16,), -1, jnp.int32)
    n_vreg = (n_sel + 15) // 16

    for k in range(_ROWCAP // 16):
        posbuf[pl.ds(k * 16, 16)] = jnp.full((16,), dummy, jnp.int32)

    def flush(off2):
        pltpu.make_async_copy(
            rowbuf.at[pl.ds(0, _ROWCAP)],
            out_hbm.at[posbuf.at[pl.ds(0, _ROWCAP)]],
            sem_out,
        ).start()
        pltpu.make_async_copy(
            rowbuf.at[pl.ds(0, _ROWCAP)],
            out_hbm.at[posbuf.at[pl.ds(0, _ROWCAP)]],
            sem_out,
        ).wait()
        for k in range(_ROWCAP // 16):
            posbuf[pl.ds(k * 16, 16)] = jnp.full((16,), dummy, jnp.int32)
        return 0

    def extract_window(w0, buf, off2):
        def record(v, offw_vec):
            rv = sel_r[pl.ds(v * 16, 16)]
            m2 = jnp.logical_and(rv >= w0, rv < w0 + _WIN)
            mi = jnp.where(m2, 1, 0).astype(jnp.int32)
            prefix = plsc.cumsum(mi)
            slots = jnp.where(m2, offw_vec + prefix - 1, _B + 16)
            plsc.store_scatter(wl_r, [slots], jnp.clip(rv - w0, 0, _WIN - 1))
            plsc.store_scatter(wl_pos, [slots], sel_pos[pl.ds(v * 16, 16)])
            return offw_vec + plsc.all_reduce_population_count(m2)

        offw_vec = lax.fori_loop(0, n_vreg, record, jnp.zeros((16,), jnp.int32))
        cnt_w = jax.lax.shift_right_logical(jnp.sum(offw_vec), 4)
        nwv = (cnt_w + 15) // 16

        def extract(t, off2):
            rloc = jnp.clip(wl_r[pl.ds(t * 16, 16)], 0, _WIN - 1)
            posv = wl_pos[pl.ds(t * 16, 16)]
            m3 = (lane + t * 16) < cnt_w
            slots = jnp.where(m3, off2 + plsc.cumsum(
                jnp.where(m3, 1, 0).astype(jnp.int32)) - 1, _ROWCAP)
            for c in range(_D):
                csplat = jnp.full((16,), c, jnp.int32)
                vals = plsc.load_gather(buf, [csplat, rloc])
                plsc.store_scatter(rowbuf, [slots, csplat], vals)
            plsc.store_scatter(posbuf, [slots], posv)
            off2 = off2 + jnp.sum(jnp.where(m3, 1, 0).astype(jnp.int32))

            @pl.when(off2 >= _ROWCAP - 16)
            def _():
                flush(off2)

            return jnp.where(off2 >= _ROWCAP - 16, 0, off2)

        return lax.fori_loop(0, nwv, extract, off2)

    def pair_body(j, off2):
        w_a = 2 * j
        w_b = 2 * j + 1
        pltpu.make_async_copy(
            emdT_hbm.at[:, pl.ds(pl.multiple_of(win_start(w_a), _WIN), _WIN)],
            win0, sem0,
        ).wait()
        off2 = extract_window(win_start(w_a), win0, off2)

        @pl.when(w_a + 2 < _NWIN)
        def _():
            fire(w_a + 2, win0, sem0)

        pltpu.make_async_copy(
            emdT_hbm.at[:, pl.ds(pl.multiple_of(win_start(w_b), _WIN), _WIN)],
            win1, sem1,
        ).wait()
        off2 = extract_window(win_start(w_b), win1, off2)

        @pl.when(w_b + 2 < _NWIN)
        def _():
            fire(w_b + 2, win1, sem1)

        return off2

    off2 = lax.fori_loop(0, _NWIN // 2, pair_body, 0)
    pltpu.make_async_copy(
        emdT_hbm.at[:, pl.ds(pl.multiple_of(win_start(_NWIN - 1), _WIN), _WIN)],
        win0, sem0,
    ).wait()
    off2 = extract_window(win_start(_NWIN - 1), win0, off2)

    @pl.when(wid == _NW - 1)
    def _():
        pltpu.make_async_copy(
            emdT_hbm.at[:, pl.ds(999424, _WIN)], win0, sem0
        ).start()
        pltpu.make_async_copy(
            emdT_hbm.at[:, pl.ds(999424, _WIN)], win0, sem0
        ).wait()
        o = extract_window(999424, win0, off2)
        pltpu.sync_copy(tailT_hbm, win0.at[:, pl.ds(0, 128)])
        o = extract_window(999872, win0, o)
        flush(o)

    @pl.when(wid != _NW - 1)
    def _():
        flush(off2)


def kernel(x, emd):
    emd_t = emd.T
    tail_t = lax.slice(emd_t, (0, _V - 128), (_D, _V))
    out_pad = _sc_scan_gather(x, emd_t, tail_t)
    return out_pad[:_B, :_D]


# speedup vs baseline: 1.1826x; 1.1302x over previous
import functools

import jax
import jax.numpy as jnp
from jax import lax
from jax.experimental import pallas as pl
from jax.experimental.pallas import tpu as pltpu
from jax.experimental.pallas import tpu_sc as plsc

_V = 1000000
_D = 32
_B = 16384

_NW = 32
_SPAN = 31232
_WBIG = 1024
_NBIG = 30
_ROWCAP = 128
_DUMMY0 = _B
_XCHUNK = 2048
_WTRASH = _B + 16
_PSHIFT = 14
_PMASK = (1 << _PSHIFT) - 1

_mesh = plsc.VectorSubcoreMesh(core_axis_name="c", subcore_axis_name="s")


@functools.partial(
    pl.kernel,
    mesh=_mesh,
    out_type=jax.ShapeDtypeStruct((_B + 32, 128), jnp.float32),
    scratch_types=[
        pltpu.VMEM((_XCHUNK,), jnp.int32),
        pltpu.VMEM((_B + 32,), jnp.int32),
        pltpu.VMEM((_B + 32,), jnp.int32),
        pltpu.VMEM((32, _WBIG), jnp.float32),
        pltpu.VMEM((32, _WBIG), jnp.float32),
        pltpu.VMEM((_ROWCAP + 1, 128), jnp.float32),
        pltpu.VMEM((_ROWCAP + 16,), jnp.int32),
        pltpu.SemaphoreType.DMA,
        pltpu.SemaphoreType.DMA,
        pltpu.SemaphoreType.DMA,
    ],
    compiler_params=pltpu.CompilerParams(needs_layout_passes=False),
)
def _sc_scan_gather(
    x_hbm, emdT_hbm, tailT_hbm, out_hbm,
    xc_v, sel_p, wl_p, win0, win1, rowbuf, posbuf,
    sem0, sem1, sem_out,
):
    wid = lax.axis_index("s") * 2 + lax.axis_index("c")
    lo = wid * _SPAN
    hi = jnp.where(wid == _NW - 1, _V, lo + _SPAN)
    lane = lax.iota(jnp.int32, 16)
    dummy = _DUMMY0 + wid

    def fire_big(w, buf, sem):
        off = pl.multiple_of(lo + w * _WBIG, _WBIG)
        pltpu.make_async_copy(
            emdT_hbm.at[:, pl.ds(off, _WBIG)], buf, sem
        ).start()

    def wait_big(w, buf, sem):
        off = pl.multiple_of(lo + w * _WBIG, _WBIG)
        pltpu.make_async_copy(
            emdT_hbm.at[:, pl.ds(off, _WBIG)], buf, sem
        ).wait()

    fire_big(0, win0, sem0)
    fire_big(1, win1, sem1)

    def chunk_body(cb, off_vec):
        pltpu.sync_copy(x_hbm.at[pl.ds(cb * _XCHUNK, _XCHUNK)], xc_v)

        def select_body(k, off_vec):
            iv = xc_v[pl.ds(k * 16, 16)]
            m = jnp.logical_and(iv >= lo, iv < hi)
            mi = jnp.where(m, 1, 0).astype(jnp.int32)
            prefix = plsc.cumsum(mi)
            slots = jnp.where(m, off_vec + prefix - 1, _WTRASH)
            packed = jax.lax.shift_left(iv - lo, _PSHIFT) | (
                lane + (cb * _XCHUNK + k * 16)
            )
            plsc.store_scatter(sel_p, [slots], packed)
            return off_vec + plsc.all_reduce_population_count(m)

        return lax.fori_loop(0, _XCHUNK // 16, select_body, off_vec)

    off_vec = lax.fori_loop(
        0, _B // _XCHUNK, chunk_body, jnp.zeros((16,), jnp.int32)
    )
    n_sel = jax.lax.shift_right_logical(jnp.sum(off_vec), 4)
    sel_p[pl.ds(n_sel, 16)] = jnp.full((16,), -1, jnp.int32)
    n_vreg = (n_sel + 15) // 16

    for k in range(_ROWCAP // 16):
        posbuf[pl.ds(k * 16, 16)] = jnp.full((16,), dummy, jnp.int32)

    def flush(off2):
        pltpu.make_async_copy(
            rowbuf.at[pl.ds(0, _ROWCAP)],
            out_hbm.at[posbuf.at[pl.ds(0, _ROWCAP)]],
            sem_out,
        ).start()
        pltpu.make_async_copy(
            rowbuf.at[pl.ds(0, _ROWCAP)],
            out_hbm.at[posbuf.at[pl.ds(0, _ROWCAP)]],
            sem_out,
        ).wait()
        for k in range(_ROWCAP // 16):
            posbuf[pl.ds(k * 16, 16)] = jnp.full((16,), dummy, jnp.int32)
        return 0

    def extract_window(w0rel, width, buf, off2):
        p_lo = w0rel << _PSHIFT
        p_hi = (w0rel + width) << _PSHIFT

        def record(v, offw_vec):
            pv = sel_p[pl.ds(v * 16, 16)]
            m2 = jnp.logical_and(pv >= p_lo, pv < p_hi)
            mi = jnp.where(m2, 1, 0).astype(jnp.int32)
            prefix = plsc.cumsum(mi)
            slots = jnp.where(m2, offw_vec + prefix - 1, _WTRASH)
            plsc.store_scatter(wl_p, [slots], pv)
            return offw_vec + plsc.all_reduce_population_count(m2)

        offw_vec = lax.fori_loop(0, n_vreg, record, jnp.zeros((16,), jnp.int32))
        cnt_w = jax.lax.shift_right_logical(jnp.sum(offw_vec), 4)
        nwv = (cnt_w + 15) // 16

        def extract(t, off2):
            pv = wl_p[pl.ds(t * 16, 16)]
            rloc = jnp.clip(
                jax.lax.shift_right_arithmetic(pv, _PSHIFT) - w0rel,
                0, width - 1,
            )
            posv = pv & _PMASK
            m3 = (lane + t * 16) < cnt_w
            mi = jnp.where(m3, 1, 0).astype(jnp.int32)
            slots = jnp.where(m3, off2 + plsc.cumsum(mi) - 1, _ROWCAP)
            for c in range(_D):
                csplat = jnp.full((16,), c, jnp.int32)
                vals = plsc.load_gather(buf, [csplat, rloc])
                plsc.store_scatter(rowbuf, [slots, csplat], vals)
            plsc.store_scatter(posbuf, [slots], posv)
            off2 = off2 + jnp.sum(mi)

            @pl.when(off2 >= _ROWCAP - 16)
            def _():
                flush(off2)

            return jnp.where(off2 >= _ROWCAP - 16, 0, off2)

        return lax.fori_loop(0, nwv, extract, off2)

    def pair_body(j, off2):
        w_a = 2 * j
        w_b = 2 * j + 1
        wait_big(w_a, win0, sem0)
        off2 = extract_window(w_a * _WBIG, _WBIG, win0, off2)

        @pl.when(w_a + 2 < _NBIG)
        def _():
            fire_big(w_a + 2, win0, sem0)

        wait_big(w_b, win1, sem1)
        off2 = extract_window(w_b * _WBIG, _WBIG, win1, off2)

        @pl.when(w_b + 2 < _NBIG)
        def _():
            fire_big(w_b + 2, win1, sem1)

        return off2

    off2 = lax.fori_loop(0, _NBIG // 2, pair_body, 0)

    pltpu.make_async_copy(
        emdT_hbm.at[:, pl.ds(pl.multiple_of(lo + _NBIG * _WBIG, 512), 512)],
        win0.at[:, pl.ds(0, 512)], sem0,
    ).start()
    pltpu.make_async_copy(
        emdT_hbm.at[:, pl.ds(pl.multiple_of(lo + _NBIG * _WBIG, 512), 512)],
        win0.at[:, pl.ds(0, 512)], sem0,
    ).wait()
    off2 = extract_window(_NBIG * _WBIG, 512, win0, off2)

    @pl.when(wid == _NW - 1)
    def _():
        pltpu.make_async_copy(
            emdT_hbm.at[:, pl.ds(999424, 512)], win0.at[:, pl.ds(0, 512)],
            sem0,
        ).start()
        pltpu.make_async_copy(
            emdT_hbm.at[:, pl.ds(999424, 512)], win0.at[:, pl.ds(0, 512)],
            sem0,
        ).wait()
        o = extract_window(_SPAN, 512, win0, off2)
        pltpu.sync_copy(tailT_hbm, win0.at[:, pl.ds(0, 128)])
        o = extract_window(999872 - 968192, 512, win0, o)
        flush(o)

    @pl.when(wid != _NW - 1)
    def _():
        flush(off2)


def kernel(x, emd):
    emd_t = emd.T
    tail_t = lax.slice(emd_t, (0, _V - 128), (_D, _V))
    out_pad = _sc_scan_gather(x, emd_t, tail_t)
    return out_pad[:_B, :_D]


# speedup vs baseline: 1.1828x; 1.0002x over previous
import functools

import jax
import jax.numpy as jnp
from jax import lax
from jax.experimental import pallas as pl
from jax.experimental.pallas import tpu as pltpu
from jax.experimental.pallas import tpu_sc as plsc

_V = 1000000
_D = 32
_B = 16384

_NW = 32
_SPAN = 31232
_WBIG = 1024
_NBIG = 30
_ROWCAP = 128
_DUMMY0 = _B
_XCHUNK = 2048
_WTRASH = _B + 16
_PSHIFT = 14
_PMASK = (1 << _PSHIFT) - 1

_mesh = plsc.VectorSubcoreMesh(core_axis_name="c", subcore_axis_name="s")


@functools.partial(
    pl.kernel,
    mesh=_mesh,
    out_type=jax.ShapeDtypeStruct((_B + 32, 128), jnp.float32),
    scratch_types=[
        pltpu.VMEM((_XCHUNK,), jnp.int32),
        pltpu.VMEM((_B + 32,), jnp.int32),
        pltpu.VMEM((_B + 32,), jnp.int32),
        pltpu.VMEM((32, _WBIG), jnp.float32),
        pltpu.VMEM((32, _WBIG), jnp.float32),
        pltpu.VMEM((_ROWCAP + 1, 128), jnp.float32),
        pltpu.VMEM((_ROWCAP + 16,), jnp.int32),
        pltpu.SemaphoreType.DMA,
        pltpu.SemaphoreType.DMA,
        pltpu.SemaphoreType.DMA,
    ],
    compiler_params=pltpu.CompilerParams(needs_layout_passes=False),
)
def _sc_scan_gather(
    x_hbm, emdT_hbm, tailT_hbm, out_hbm,
    xc_v, sel_p, wl_p, win0, win1, rowbuf, posbuf,
    sem0, sem1, sem_out,
):
    wid = lax.axis_index("s") * 2 + lax.axis_index("c")
    lo = wid * _SPAN
    hi = jnp.where(wid == _NW - 1, _V, lo + _SPAN)
    lane = lax.iota(jnp.int32, 16)
    dummy = _DUMMY0 + wid

    def fire_big(w, buf, sem):
        off = pl.multiple_of(lo + w * _WBIG, _WBIG)
        pltpu.make_async_copy(
            emdT_hbm.at[:, pl.ds(off, _WBIG)], buf, sem
        ).start()

    def wait_big(w, buf, sem):
        off = pl.multiple_of(lo + w * _WBIG, _WBIG)
        pltpu.make_async_copy(
            emdT_hbm.at[:, pl.ds(off, _WBIG)], buf, sem
        ).wait()

    fire_big(0, win0, sem0)
    fire_big(1, win1, sem1)

    def chunk_body(cb, off_vec):
        pltpu.sync_copy(x_hbm.at[pl.ds(cb * _XCHUNK, _XCHUNK)], xc_v)

        def select_one(k, off_vec):
            iv = xc_v[pl.ds(k * 16, 16)]
            m = jnp.logical_and(iv >= lo, iv < hi)
            mi = jnp.where(m, 1, 0).astype(jnp.int32)
            prefix = plsc.cumsum(mi)
            slots = jnp.where(m, off_vec + prefix - 1, _WTRASH)
            packed = jax.lax.shift_left(iv - lo, _PSHIFT) | (
                lane + (cb * _XCHUNK + k * 16)
            )
            plsc.store_scatter(sel_p, [slots], packed)
            return off_vec + plsc.all_reduce_population_count(m)

        def select_body(k2, off_vec):
            off_vec = select_one(2 * k2, off_vec)
            return select_one(2 * k2 + 1, off_vec)

        return lax.fori_loop(0, _XCHUNK // 32, select_body, off_vec)

    off_vec = lax.fori_loop(
        0, _B // _XCHUNK, chunk_body, jnp.zeros((16,), jnp.int32)
    )
    n_sel = jax.lax.shift_right_logical(jnp.sum(off_vec), 4)
    sel_p[pl.ds(n_sel, 16)] = jnp.full((16,), -1, jnp.int32)
    sel_p[pl.ds(n_sel + 16, 16)] = jnp.full((16,), -1, jnp.int32)
    n_vreg = (n_sel + 15) // 16

    for k in range(_ROWCAP // 16):
        posbuf[pl.ds(k * 16, 16)] = jnp.full((16,), dummy, jnp.int32)

    def flush(off2):
        pltpu.make_async_copy(
            rowbuf.at[pl.ds(0, _ROWCAP)],
            out_hbm.at[posbuf.at[pl.ds(0, _ROWCAP)]],
            sem_out,
        ).start()
        pltpu.make_async_copy(
            rowbuf.at[pl.ds(0, _ROWCAP)],
            out_hbm.at[posbuf.at[pl.ds(0, _ROWCAP)]],
            sem_out,
        ).wait()
        for k in range(_ROWCAP // 16):
            posbuf[pl.ds(k * 16, 16)] = jnp.full((16,), dummy, jnp.int32)
        return 0

    def extract_window(w0rel, width, buf, off2):
        p_lo = w0rel << _PSHIFT
        p_hi = (w0rel + width) << _PSHIFT

        def record_one(v, offw_vec):
            pv = sel_p[pl.ds(v * 16, 16)]
            m2 = jnp.logical_and(pv >= p_lo, pv < p_hi)
            mi = jnp.where(m2, 1, 0).astype(jnp.int32)
            prefix = plsc.cumsum(mi)
            slots = jnp.where(m2, offw_vec + prefix - 1, _WTRASH)
            plsc.store_scatter(wl_p, [slots], pv)
            return offw_vec + plsc.all_reduce_population_count(m2)

        def record(v2, offw_vec):
            offw_vec = record_one(2 * v2, offw_vec)
            return record_one(2 * v2 + 1, offw_vec)

        offw_vec = lax.fori_loop(
            0, (n_vreg + 1) // 2, record, jnp.zeros((16,), jnp.int32)
        )
        cnt_w = jax.lax.shift_right_logical(jnp.sum(offw_vec), 4)
        nwv = (cnt_w + 15) // 16

        def extract(t, off2):
            pv = wl_p[pl.ds(t * 16, 16)]
            rloc = jnp.clip(
                jax.lax.shift_right_arithmetic(pv, _PSHIFT) - w0rel,
                0, width - 1,
            )
            posv = pv & _PMASK
            m3 = (lane + t * 16) < cnt_w
            mi = jnp.where(m3, 1, 0).astype(jnp.int32)
            slots = jnp.where(m3, off2 + plsc.cumsum(mi) - 1, _ROWCAP)
            for c in range(_D):
                csplat = jnp.full((16,), c, jnp.int32)
                vals = plsc.load_gather(buf, [csplat, rloc])
                plsc.store_scatter(rowbuf, [slots, csplat], vals)
            plsc.store_scatter(posbuf, [slots], posv)
            off2 = off2 + jnp.sum(mi)

            @pl.when(off2 >= _ROWCAP - 16)
            def _():
                flush(off2)

            return jnp.where(off2 >= _ROWCAP - 16, 0, off2)

        return lax.fori_loop(0, nwv, extract, off2)

    def pair_body(j, off2):
        w_a = 2 * j
        w_b = 2 * j + 1
        wait_big(w_a, win0, sem0)
        off2 = extract_window(w_a * _WBIG, _WBIG, win0, off2)

        @pl.when(w_a + 2 < _NBIG)
        def _():
            fire_big(w_a + 2, win0, sem0)

        wait_big(w_b, win1, sem1)
        off2 = extract_window(w_b * _WBIG, _WBIG, win1, off2)

        @pl.when(w_b + 2 < _NBIG)
        def _():
            fire_big(w_b + 2, win1, sem1)

        return off2

    off2 = lax.fori_loop(0, _NBIG // 2, pair_body, 0)

    pltpu.make_async_copy(
        emdT_hbm.at[:, pl.ds(pl.multiple_of(lo + _NBIG * _WBIG, 512), 512)],
        win0.at[:, pl.ds(0, 512)], sem0,
    ).start()
    pltpu.make_async_copy(
        emdT_hbm.at[:, pl.ds(pl.multiple_of(lo + _NBIG * _WBIG, 512), 512)],
        win0.at[:, pl.ds(0, 512)], sem0,
    ).wait()
    off2 = extract_window(_NBIG * _WBIG, 512, win0, off2)

    @pl.when(wid == _NW - 1)
    def _():
        pltpu.make_async_copy(
            emdT_hbm.at[:, pl.ds(999424, 512)], win0.at[:, pl.ds(0, 512)],
            sem0,
        ).start()
        pltpu.make_async_copy(
            emdT_hbm.at[:, pl.ds(999424, 512)], win0.at[:, pl.ds(0, 512)],
            sem0,
        ).wait()
        o = extract_window(_SPAN, 512, win0, off2)
        pltpu.sync_copy(tailT_hbm, win0.at[:, pl.ds(0, 128)])
        o = extract_window(999872 - 968192, 512, win0, o)
        flush(o)

    @pl.when(wid != _NW - 1)
    def _():
        flush(off2)


def kernel(x, emd):
    emd_t = emd.T
    tail_t = lax.slice(emd_t, (0, _V - 128), (_D, _V))
    out_pad = _sc_scan_gather(x, emd_t, tail_t)
    return out_pad[:_B, :_D]
